# baseline (speedup 1.0000x reference)
import numpy as np
import jax
import jax.numpy as jnp
from jax.experimental import pallas as pl
from jax.experimental.pallas import tpu as pltpu

_VBLK = 2048
_TINY = np.float32(np.finfo(np.float32).tiny)
_IMAX = np.int32(np.iinfo(np.int32).max)


def _gumbel_bits(flat_u32):
    k0 = np.uint32(0)
    k1 = np.uint32(42)
    ks2 = np.uint32(0 ^ 42 ^ 0x1BD11BDA)

    def rot(x, r):
        return (x << np.uint32(r)) | (x >> np.uint32(32 - r))

    def rounds(x0, x1, rots):
        for r in rots:
            x0 = x0 + x1
            x1 = rot(x1, r) ^ x0
        return x0, x1

    x0 = jnp.zeros_like(flat_u32) + k0
    x1 = flat_u32 + k1
    x0, x1 = rounds(x0, x1, (13, 15, 26, 6))
    x0 = x0 + k1
    x1 = x1 + np.uint32(ks2 + np.uint32(1))
    x0, x1 = rounds(x0, x1, (17, 29, 16, 24))
    x0 = x0 + ks2
    x1 = x1 + np.uint32(k0 + np.uint32(2))
    x0, x1 = rounds(x0, x1, (13, 15, 26, 6))
    x0 = x0 + k0
    x1 = x1 + np.uint32(k1 + np.uint32(3))
    x0, x1 = rounds(x0, x1, (17, 29, 16, 24))
    x0 = x0 + k1
    x1 = x1 + np.uint32(ks2 + np.uint32(4))
    x0, x1 = rounds(x0, x1, (13, 15, 26, 6))
    x0 = x0 + ks2
    x1 = x1 + np.uint32(k0 + np.uint32(5))
    bits = x0 ^ x1

    fbits = (bits >> np.uint32(9)) | np.uint32(0x3F800000)
    floats = jax.lax.bitcast_convert_type(fbits, jnp.float32) - np.float32(1.0)
    u = jnp.maximum(floats, _TINY)
    return -jnp.log(-jnp.log(u))


def _body(nblk, bsz, vocab, logits_ref, mask_ref, out_logits_ref, out_ints_ref,
          bv_ref, bi_ref):
    v = pl.program_id(0)
    xm = logits_ref[:, 0, 0, :] + mask_ref[0, :][None, :]
    out_logits_ref[:, :] = xm

    col = jax.lax.broadcasted_iota(jnp.int32, (bsz, _VBLK), 1) + v * _VBLK
    row = jax.lax.broadcasted_iota(jnp.int32, (bsz, _VBLK), 0)
    flat = (row * vocab + col).astype(jnp.uint32)
    tot = xm + _gumbel_bits(flat)
    tot = jnp.where(col < vocab, tot, -jnp.inf)

    bmax = jnp.max(tot, axis=1, keepdims=True)
    barg = jnp.min(jnp.where(tot == bmax, col, _IMAX), axis=1, keepdims=True)
    bmax_b = jnp.broadcast_to(bmax, (bsz, 128))
    barg_b = jnp.broadcast_to(barg, (bsz, 128))

    @pl.when(v == 0)
    def _():
        bv_ref[:, :] = bmax_b
        bi_ref[:, :] = barg_b

    @pl.when(v > 0)
    def _():
        better = jnp.broadcast_to(bmax > bv_ref[:, 0:1], (bsz, 128))
        bv_ref[:, :] = jnp.where(better, bmax_b, bv_ref[:, :])
        bi_ref[:, :] = jnp.where(better, barg_b, bi_ref[:, :])

    @pl.when(v == nblk - 1)
    def _():
        out_ints_ref[:, :] = bi_ref[:, :]


def _build(bsz, steps, vocab, interpret=False):
    import functools
    nblk = pl.cdiv(vocab, _VBLK)
    return pl.pallas_call(
        functools.partial(_body, nblk, bsz, vocab),
        grid=(nblk,),
        in_specs=[
            pl.BlockSpec((bsz, 1, 1, _VBLK), lambda v: (0, steps - 1, 0, v)),
            pl.BlockSpec((1, _VBLK), lambda v: (0, v)),
        ],
        out_specs=[
            pl.BlockSpec((bsz, _VBLK), lambda v: (0, v)),
            pl.BlockSpec((bsz, 128), lambda v: (0, 0)),
        ],
        out_shape=[
            jax.ShapeDtypeStruct((bsz, vocab), jnp.float32),
            jax.ShapeDtypeStruct((bsz, 128), jnp.int32),
        ],
        scratch_shapes=[
            pltpu.VMEM((bsz, 128), jnp.float32),
            pltpu.VMEM((bsz, 128), jnp.int32),
        ],
        interpret=interpret,
    )


def kernel(logits, prediction_mask):
    bsz, steps, vocab = logits.shape
    out_logits, out_ints = _build(bsz, steps, vocab)(
        logits.reshape(bsz, steps, 1, vocab), prediction_mask.reshape(1, vocab))
    return out_ints[:, 0], out_logits
